# baseline (speedup 1.0000x reference)
import functools

import jax
import jax.numpy as jnp
from jax import lax
from jax.experimental import pallas as pl
from jax.experimental.pallas import tpu as pltpu, tpu_sc as plsc


@functools.lru_cache(maxsize=None)
def _make_gather(V, D, B):
    info = plsc.get_sparse_core_info()
    NC, NS, L = info.num_cores, info.num_subcores, info.num_lanes
    NW = NC * NS
    assert B % (8 * NW) == 0
    b_per_w = B // NW
    G = 128
    HD = D // 2
    mesh = plsc.VectorSubcoreMesh(core_axis_name="c", subcore_axis_name="s")

    @functools.partial(
        pl.kernel,
        mesh=mesh,
        out_type=jax.ShapeDtypeStruct((D, B), jnp.float32),
        scratch_types=[
            pltpu.VMEM((b_per_w,), jnp.int32),
            pltpu.VMEM((D, b_per_w), jnp.int32),
            pltpu.VMEM((D, b_per_w), jnp.float32),
            pltpu.SemaphoreType.DMA,
        ],
        compiler_params=pltpu.CompilerParams(use_tc_tiling_on_sc=False),
    )
    def k(tableT_hbm, idx_hbm, out_hbm, idx_v, fidx_v, rows_v, sem):
        wid = lax.axis_index("s") * NC + lax.axis_index("c")
        base = wid * b_per_w
        pltpu.sync_copy(idx_hbm.at[pl.ds(base, b_per_w)], idx_v)
        copies = []
        for c in range(D):
            for q in range(b_per_w // G):
                copies.append(
                    pltpu.async_copy(
                        tableT_hbm.at[c].at[idx_v.at[pl.ds(q * G, G)]],
                        rows_v.at[c, pl.ds(q * G, G)],
                        sem,
                    )
                )
        for cp in copies:
            cp.wait()
        pltpu.sync_copy(rows_v, out_hbm.at[:, pl.ds(base, b_per_w)])

    return k


@jax.jit
def kernel(id_indices, embeddings):
    B = id_indices.shape[0]
    V, D = embeddings.shape
    k = _make_gather(V, D, B)
    out_t = k(embeddings.T, id_indices.astype(jnp.int32))
    return out_t.T


# speedup vs baseline: 1.1282x; 1.1282x over previous
import functools

import jax
import jax.numpy as jnp
from jax import lax
from jax.experimental import pallas as pl
from jax.experimental.pallas import tpu as pltpu, tpu_sc as plsc


@functools.lru_cache(maxsize=None)
def _make_gather(VP, D, B):
    info = plsc.get_sparse_core_info()
    NC, NS, L = info.num_cores, info.num_subcores, info.num_lanes
    NW = NC * NS
    assert B % (8 * NW) == 0
    b_per_w = B // NW
    G = 128
    mesh = plsc.VectorSubcoreMesh(core_axis_name="c", subcore_axis_name="s")

    @functools.partial(
        pl.kernel,
        mesh=mesh,
        out_type=jax.ShapeDtypeStruct((D, B), jnp.float32),
        scratch_types=[
            pltpu.VMEM((b_per_w,), jnp.int32),
            pltpu.VMEM((D, b_per_w), jnp.float32),
            pltpu.SemaphoreType.DMA,
        ],
        compiler_params=pltpu.CompilerParams(use_tc_tiling_on_sc=False),
    )
    def k(tableT_hbm, idx_hbm, out_hbm, idx_v, rows_v, sem):
        wid = lax.axis_index("s") * NC + lax.axis_index("c")
        base = wid * b_per_w
        pltpu.sync_copy(idx_hbm.at[pl.ds(base, b_per_w)], idx_v)
        copies = []
        for c in range(D):
            for q in range(b_per_w // G):
                copies.append(
                    pltpu.async_copy(
                        tableT_hbm.at[c].at[idx_v.at[pl.ds(q * G, G)]],
                        rows_v.at[c, pl.ds(q * G, G)],
                        sem,
                    )
                )
        for cp in copies:
            cp.wait()
        pltpu.sync_copy(rows_v, out_hbm.at[:, pl.ds(base, b_per_w)])

    return k


@jax.jit
def kernel(id_indices, embeddings):
    B = id_indices.shape[0]
    V, D = embeddings.shape
    VP = (V + 127) // 128 * 128
    tableT = jnp.pad(embeddings.T, ((0, 0), (0, VP - V)))
    k = _make_gather(VP, D, B)
    out_t = k(tableT, id_indices.astype(jnp.int32))
    return out_t.T


# speedup vs baseline: 18.8697x; 16.7250x over previous
import functools

import jax
import jax.numpy as jnp
from jax import lax
from jax.experimental import pallas as pl
from jax.experimental.pallas import tpu as pltpu, tpu_sc as plsc

_L = 128


@functools.lru_cache(maxsize=None)
def _make_relayout(V, D):
    info = plsc.get_sparse_core_info()
    NC, NS = info.num_cores, info.num_subcores
    NW = NC * NS
    G8 = D // 8
    n_full = V // _L
    n_blocks = n_full // 8
    rem = n_full - n_blocks * 8
    R = (n_full + 1 + 7) // 8 * 8
    n_it = (n_blocks - 1) // NW + 1
    mesh = plsc.VectorSubcoreMesh(core_axis_name="c", subcore_axis_name="s")

    @functools.partial(
        pl.kernel,
        mesh=mesh,
        out_type=jax.ShapeDtypeStruct((G8, 8, R, _L), jnp.float32),
        scratch_types=[
            pltpu.VMEM((2, G8, 8, 8, _L), jnp.float32),
            pltpu.SemaphoreType.DMA,
            pltpu.SemaphoreType.DMA,
        ],
    )
    def k(table_hbm, tail_hbm, out_hbm, buf_v, in_sem, out_sem):
        wid = lax.axis_index("s") * NC + lax.axis_index("c")

        def in_descs(b, slot):
            ds = []
            for j in range(8):
                start = pl.multiple_of((b * 8 + j) * _L, _L)
                ds.append(
                    pltpu.make_async_copy(
                        table_hbm.at[:, :, pl.ds(start, _L)],
                        buf_v.at[slot, :, :, j, :],
                        in_sem,
                    )
                )
            return ds

        def out_desc(b, slot):
            start = pl.multiple_of(b * 8, 8)
            return pltpu.make_async_copy(
                buf_v.at[slot],
                out_hbm.at[:, :, pl.ds(start, 8), :],
                out_sem,
            )

        b0 = wid

        @pl.when(b0 < n_blocks)
        def _():
            for d in in_descs(b0, 0):
                d.start()

        def body(it, _):
            b = wid + it * NW
            b_next = wid + (it + 1) * NW
            b_prev = wid + (it - 1) * NW

            @pl.when(jnp.logical_and(it >= 1, b_prev < n_blocks))
            def _():
                out_desc(b_prev, lax.rem(it - 1, 2)).wait()

            @pl.when(b_next < n_blocks)
            def _():
                for d in in_descs(b_next, lax.rem(it + 1, 2)):
                    d.start()

            @pl.when(b < n_blocks)
            def _():
                slot = lax.rem(it, 2)
                for d in in_descs(b, slot):
                    d.wait()
                out_desc(b, slot).start()

            return 0

        lax.fori_loop(0, n_it, body, 0)
        b_last = wid + (n_it - 1) * NW

        @pl.when(b_last < n_blocks)
        def _():
            out_desc(b_last, lax.rem(n_it - 1, 2)).wait()

        @pl.when(wid == NW - 1)
        def _():
            tds = []
            for j in range(rem):
                start = pl.multiple_of((n_blocks * 8 + j) * _L, _L)
                d = pltpu.make_async_copy(
                    table_hbm.at[:, :, pl.ds(start, _L)],
                    buf_v.at[0, :, :, j, :],
                    in_sem,
                )
                d.start()
                tds.append(d)
            dt = pltpu.make_async_copy(
                tail_hbm, buf_v.at[0, :, :, rem, :], in_sem
            )
            dt.start()
            for d in tds:
                d.wait()
            dt.wait()
            dlast = pltpu.make_async_copy(
                buf_v.at[0],
                out_hbm.at[:, :, pl.ds(n_blocks * 8, 8), :],
                out_sem,
            )
            dlast.start()
            dlast.wait()

    return k


@functools.lru_cache(maxsize=None)
def _make_gather(VP, D, B):
    info = plsc.get_sparse_core_info()
    NC, NS, L = info.num_cores, info.num_subcores, info.num_lanes
    NW = NC * NS
    assert B % (8 * NW) == 0
    b_per_w = B // NW
    G = 128
    mesh = plsc.VectorSubcoreMesh(core_axis_name="c", subcore_axis_name="s")

    @functools.partial(
        pl.kernel,
        mesh=mesh,
        out_type=jax.ShapeDtypeStruct((D, B), jnp.float32),
        scratch_types=[
            pltpu.VMEM((b_per_w,), jnp.int32),
            pltpu.VMEM((D, b_per_w), jnp.float32),
            pltpu.SemaphoreType.DMA,
        ],
        compiler_params=pltpu.CompilerParams(use_tc_tiling_on_sc=False),
    )
    def k(tableT_hbm, idx_hbm, out_hbm, idx_v, rows_v, sem):
        wid = lax.axis_index("s") * NC + lax.axis_index("c")
        base = wid * b_per_w
        pltpu.sync_copy(idx_hbm.at[pl.ds(base, b_per_w)], idx_v)
        copies = []
        for c in range(D):
            for q in range(b_per_w // G):
                copies.append(
                    pltpu.async_copy(
                        tableT_hbm.at[c].at[idx_v.at[pl.ds(q * G, G)]],
                        rows_v.at[c, pl.ds(q * G, G)],
                        sem,
                    )
                )
        for cp in copies:
            cp.wait()
        pltpu.sync_copy(rows_v, out_hbm.at[:, pl.ds(base, b_per_w)])

    return k


@jax.jit
def kernel(id_indices, embeddings):
    B = id_indices.shape[0]
    V, D = embeddings.shape
    n_full = V // _L
    R = (n_full + 1 + 7) // 8 * 8
    VP = R * _L

    table3 = embeddings.T.reshape(D // 8, 8, V)
    tail = embeddings[n_full * _L :, :]
    tail_p = jnp.pad(tail.T, ((0, 0), (0, _L - tail.shape[0])))
    tail3 = tail_p.reshape(D // 8, 8, _L)

    k1 = _make_relayout(V, D)
    lin4 = k1(table3, tail3)
    lin = lin4.reshape(D, VP)

    k2 = _make_gather(VP, D, B)
    out_t = k2(lin, id_indices.astype(jnp.int32))
    return out_t.T


# speedup vs baseline: 25.6406x; 1.3588x over previous
import functools

import jax
import jax.numpy as jnp
from jax import lax
from jax.experimental import pallas as pl
from jax.experimental.pallas import tpu as pltpu, tpu_sc as plsc

_L = 128


@functools.lru_cache(maxsize=None)
def _make_fused(V, D, B):
    info = plsc.get_sparse_core_info()
    NC, NS, L = info.num_cores, info.num_subcores, info.num_lanes
    NW = NC * NS
    assert D == NW
    W = 32768
    LOGW = 15
    n_win = (V + W - 1) // W
    n_full = (V // _L) * _L
    NB = n_win + 1
    n_chunks = B // L
    mesh = plsc.VectorSubcoreMesh(core_axis_name="c", subcore_axis_name="s")

    @functools.partial(
        pl.kernel,
        mesh=mesh,
        out_type=jax.ShapeDtypeStruct((D, B), jnp.float32),
        scratch_types=[
            pltpu.VMEM((B,), jnp.int32),
            pltpu.VMEM((B,), jnp.int32),
            pltpu.VMEM((NB,), jnp.int32),
            pltpu.VMEM((NB,), jnp.int32),
            pltpu.VMEM((NB,), jnp.int32),
            pltpu.VMEM((1, 1, W), jnp.float32),
            pltpu.VMEM((1, 1, W), jnp.float32),
            pltpu.VMEM((1, B), jnp.float32),
            pltpu.SemaphoreType.DMA,
            pltpu.SemaphoreType.DMA,
        ],
        compiler_params=pltpu.CompilerParams(needs_layout_passes=False),
    )
    def k(table_hbm, tail_hbm, idx_hbm, out_hbm,
          idx_v, pk_v, starts_v, cur_v, hist_v, win0_v, win1_v,
          row_v, idx_sem, sem):
        wid = lax.axis_index("s") * NC + lax.axis_index("c")
        g = lax.div(wid, 8)
        s = lax.rem(wid, 8)
        wins = (win0_v, win1_v)

        def win_in_descs(w, slot):
            lanes = min(n_full - w * W, W)
            ds = [
                pltpu.make_async_copy(
                    table_hbm.at[pl.ds(g, 1), pl.ds(s, 1), pl.ds(w * W, lanes)],
                    wins[slot].at[:, :, pl.ds(0, lanes)],
                    sem,
                )
            ]
            if lanes < W:
                ds.append(
                    pltpu.make_async_copy(
                        tail_hbm.at[pl.ds(g, 1), pl.ds(s, 1), :],
                        wins[slot].at[:, :, pl.ds(lanes, _L)],
                        sem,
                    )
                )
            return ds

        pltpu.make_async_copy(idx_hbm, idx_v, idx_sem).start()
        first = win_in_descs(0, 0)
        for d in first:
            d.start()
        pltpu.make_async_copy(idx_hbm, idx_v, idx_sem).wait()

        zl = jnp.zeros((L,), jnp.int32)
        hist_v[pl.ds(0, L)] = zl
        hist_v[pl.ds(L, L)] = zl

        def hist_body(c, _):
            v = idx_v[pl.ds(c * L, L)]
            b = jax.lax.shift_right_logical(v, LOGW)
            cnt, last = plsc.scan_count(b)
            plsc.addupdate_scatter(hist_v, [b], cnt, mask=last)
            return 0

        lax.fori_loop(0, n_chunks, hist_body, 0)

        c0 = hist_v[pl.ds(0, L)]
        s0 = plsc.cumsum(c0)
        e0 = s0 - c0
        starts_v[pl.ds(0, L)] = e0
        cur_v[pl.ds(0, L)] = e0
        carry = jnp.sum(c0)
        c1 = hist_v[pl.ds(L, L)]
        s1 = plsc.cumsum(c1) + carry
        e1 = s1 - c1
        starts_v[pl.ds(L, L)] = e1
        cur_v[pl.ds(L, L)] = e1

        iota = lax.iota(jnp.int32, L)

        def _extract(vec, lane):
            return jnp.sum(jnp.where(iota == lane, vec, 0))

        def scat_body(c, _):
            v = idx_v[pl.ds(c * L, L)]
            p = iota + c * L
            b = jax.lax.shift_right_logical(v, LOGW)
            cnt, last = plsc.scan_count(b)
            base = plsc.load_gather(cur_v, [b])
            slot = base + cnt - 1
            lidx = v - jax.lax.shift_left(b, LOGW)
            pk = jax.lax.shift_left(lidx, 14) + p
            plsc.store_scatter(pk_v, [slot], pk)
            plsc.addupdate_scatter(cur_v, [b], cnt, mask=last)
            return 0

        lax.fori_loop(0, n_chunks, scat_body, 0)

        zero16 = jnp.zeros((L,), jnp.int32)
        for w in range(n_win):
            if w + 1 < n_win:
                for d in win_in_descs(w + 1, (w + 1) % 2):
                    d.start()
            for d in win_in_descs(w, w % 2):
                d.wait()
            lo = _extract(starts_v[pl.ds((w // L) * L, L)], w % L)
            hi = _extract(starts_v[pl.ds(((w + 1) // L) * L, L)], (w + 1) % L)
            k0 = jax.lax.shift_right_logical(lo, 4)
            k1 = jax.lax.shift_right_logical(hi + (L - 1), 4)

            def gat_body(c, _, w=w):
                base = c * L
                pk = pk_v[pl.ds(pl.multiple_of(base, L), L)]
                pos_i = iota + base
                m = jnp.logical_and(pos_i >= lo, pos_i < hi)
                lidx = jax.lax.shift_right_logical(pk, 14)
                p = jax.lax.bitwise_and(pk, (1 << 14) - 1)
                vals = plsc.load_gather(
                    wins[w % 2], [zero16, zero16, lidx], mask=m
                )
                plsc.store_scatter(row_v, [zero16, p], vals, mask=m)
                return 0

            lax.fori_loop(k0, k1, gat_body, 0)

        pltpu.sync_copy(row_v, out_hbm.at[pl.ds(wid, 1)])

    return k


@jax.jit
def kernel(id_indices, embeddings):
    B = id_indices.shape[0]
    V, D = embeddings.shape
    n_full = (V // _L) * _L

    table3 = embeddings.T.reshape(D // 8, 8, V)
    tail = embeddings[n_full:, :]
    tail_p = jnp.pad(tail.T, ((0, 0), (0, _L - tail.shape[0])))
    tail3 = tail_p.reshape(D // 8, 8, _L)

    k = _make_fused(V, D, B)
    out_t = k(table3, tail3, id_indices.astype(jnp.int32))
    return out_t.T


# speedup vs baseline: 28.2305x; 1.1010x over previous
import functools

import jax
import jax.numpy as jnp
from jax import lax
from jax.experimental import pallas as pl
from jax.experimental.pallas import tpu as pltpu, tpu_sc as plsc

_L = 128


@functools.lru_cache(maxsize=None)
def _make_fused(V, D, B):
    info = plsc.get_sparse_core_info()
    NC, NS, L = info.num_cores, info.num_subcores, info.num_lanes
    NW = NC * NS
    assert D == NW
    W = 32768
    LOGW = 15
    n_win = (V + W - 1) // W
    n_full = (V // _L) * _L
    NB = n_win + 1
    n_chunks = B // L
    mesh = plsc.VectorSubcoreMesh(core_axis_name="c", subcore_axis_name="s")

    @functools.partial(
        pl.kernel,
        mesh=mesh,
        out_type=jax.ShapeDtypeStruct((D, B), jnp.float32),
        scratch_types=[
            pltpu.VMEM((B,), jnp.int32),
            pltpu.VMEM((B,), jnp.int32),
            pltpu.VMEM((NB,), jnp.int32),
            pltpu.VMEM((NB,), jnp.int32),
            pltpu.VMEM((NB,), jnp.int32),
            pltpu.VMEM((1, 1, W), jnp.float32),
            pltpu.VMEM((1, 1, W), jnp.float32),
            pltpu.VMEM((1, B), jnp.float32),
            pltpu.SemaphoreType.DMA,
            pltpu.SemaphoreType.DMA,
        ],
        compiler_params=pltpu.CompilerParams(needs_layout_passes=False),
    )
    def k(table_hbm, tail_hbm, idx_hbm, out_hbm,
          idx_v, pk_v, starts_v, cur_v, hist_v, win0_v, win1_v,
          row_v, idx_sem, sem):
        wid = lax.axis_index("s") * NC + lax.axis_index("c")
        g = lax.div(wid, 8)
        s = lax.rem(wid, 8)
        wins = (win0_v, win1_v)

        def win_in_descs(w, slot):
            lanes = min(n_full - w * W, W)
            ds = [
                pltpu.make_async_copy(
                    table_hbm.at[pl.ds(g, 1), pl.ds(s, 1), pl.ds(w * W, lanes)],
                    wins[slot].at[:, :, pl.ds(0, lanes)],
                    sem,
                )
            ]
            if lanes < W:
                ds.append(
                    pltpu.make_async_copy(
                        tail_hbm.at[pl.ds(g, 1), pl.ds(s, 1), :],
                        wins[slot].at[:, :, pl.ds(lanes, _L)],
                        sem,
                    )
                )
            return ds

        pltpu.make_async_copy(idx_hbm, idx_v, idx_sem).start()
        first = win_in_descs(0, 0)
        for d in first:
            d.start()
        pltpu.make_async_copy(idx_hbm, idx_v, idx_sem).wait()

        zl = jnp.zeros((L,), jnp.int32)
        ones = jnp.ones((L,), jnp.int32)
        hist_v[pl.ds(0, L)] = zl
        hist_v[pl.ds(L, L)] = zl

        def hist_body(c, _):
            v = idx_v[pl.ds(c * L, L)]
            b = jax.lax.shift_right_logical(v, LOGW)
            plsc.addupdate_scatter(hist_v, [b], ones)
            return 0

        lax.fori_loop(0, n_chunks, hist_body, 0)

        c0 = hist_v[pl.ds(0, L)]
        s0 = plsc.cumsum(c0)
        e0 = s0 - c0
        starts_v[pl.ds(0, L)] = e0
        cur_v[pl.ds(0, L)] = e0
        carry = jnp.sum(c0)
        c1 = hist_v[pl.ds(L, L)]
        s1 = plsc.cumsum(c1) + carry
        e1 = s1 - c1
        starts_v[pl.ds(L, L)] = e1
        cur_v[pl.ds(L, L)] = e1

        iota = lax.iota(jnp.int32, L)

        def _extract(vec, lane):
            return jnp.sum(jnp.where(iota == lane, vec, 0))

        def scat_body(c, _):
            v = idx_v[pl.ds(c * L, L)]
            p = iota + c * L
            b = jax.lax.shift_right_logical(v, LOGW)
            cnt, last = plsc.scan_count(b)
            base = plsc.load_gather(cur_v, [b])
            slot = base + cnt - 1
            lidx = v - jax.lax.shift_left(b, LOGW)
            pk = jax.lax.shift_left(lidx, 14) + p
            plsc.store_scatter(pk_v, [slot], pk)
            plsc.addupdate_scatter(cur_v, [b], cnt, mask=last)
            return 0

        lax.fori_loop(0, n_chunks, scat_body, 0)

        zero16 = jnp.zeros((L,), jnp.int32)
        for w in range(n_win):
            if w + 1 < n_win:
                for d in win_in_descs(w + 1, (w + 1) % 2):
                    d.start()
            for d in win_in_descs(w, w % 2):
                d.wait()
            lo = _extract(starts_v[pl.ds((w // L) * L, L)], w % L)
            hi = _extract(starts_v[pl.ds(((w + 1) // L) * L, L)], (w + 1) % L)
            k0 = jax.lax.shift_right_logical(lo, 4)
            k1 = jax.lax.shift_right_logical(hi + (L - 1), 4)

            def gat_body(c, _, w=w):
                base = c * L
                pk = pk_v[pl.ds(pl.multiple_of(base, L), L)]
                pos_i = iota + base
                m = jnp.logical_and(pos_i >= lo, pos_i < hi)
                lidx = jax.lax.shift_right_logical(pk, 14)
                p = jax.lax.bitwise_and(pk, (1 << 14) - 1)
                vals = plsc.load_gather(
                    wins[w % 2], [zero16, zero16, lidx], mask=m
                )
                plsc.store_scatter(row_v, [zero16, p], vals, mask=m)
                return 0

            lax.fori_loop(k0, k1, gat_body, 0)

        pltpu.sync_copy(row_v, out_hbm.at[pl.ds(wid, 1)])

    return k


@jax.jit
def kernel(id_indices, embeddings):
    B = id_indices.shape[0]
    V, D = embeddings.shape
    n_full = (V // _L) * _L

    table3 = embeddings.T.reshape(D // 8, 8, V)
    tail = embeddings[n_full:, :]
    tail_p = jnp.pad(tail.T, ((0, 0), (0, _L - tail.shape[0])))
    tail3 = tail_p.reshape(D // 8, 8, _L)

    k = _make_fused(V, D, B)
    out_t = k(table3, tail3, id_indices.astype(jnp.int32))
    return out_t.T


# speedup vs baseline: 32.3957x; 1.1475x over previous
import functools

import jax
import jax.numpy as jnp
from jax import lax
from jax.experimental import pallas as pl
from jax.experimental.pallas import tpu as pltpu, tpu_sc as plsc

_L = 128


@functools.lru_cache(maxsize=None)
def _make_fused(V, D, B):
    info = plsc.get_sparse_core_info()
    NC, NS, L = info.num_cores, info.num_subcores, info.num_lanes
    NW = NC * NS
    assert D == NW
    W = 32768
    LOGW = 15
    n_win = (V + W - 1) // W
    n_full = (V // _L) * _L
    NB = n_win + 1
    n_chunks = B // L
    mesh = plsc.VectorSubcoreMesh(core_axis_name="c", subcore_axis_name="s")

    @functools.partial(
        pl.kernel,
        mesh=mesh,
        out_type=jax.ShapeDtypeStruct((D, B), jnp.float32),
        scratch_types=[
            pltpu.VMEM((B,), jnp.int32),
            pltpu.VMEM((B,), jnp.int32),
            pltpu.VMEM((NB,), jnp.int32),
            pltpu.VMEM((NB,), jnp.int32),
            pltpu.VMEM((NB,), jnp.int32),
            pltpu.VMEM((1, 1, W), jnp.float32),
            pltpu.VMEM((1, 1, W), jnp.float32),
            pltpu.VMEM((1, B), jnp.float32),
            pltpu.SemaphoreType.DMA,
            pltpu.SemaphoreType.DMA,
        ],
        compiler_params=pltpu.CompilerParams(needs_layout_passes=False),
    )
    def k(table_hbm, tail_hbm, idx_hbm, out_hbm,
          idx_v, pk_v, starts_v, cur_v, hist_v, win0_v, win1_v,
          row_v, idx_sem, sem):
        wid = lax.axis_index("s") * NC + lax.axis_index("c")
        g = lax.div(wid, 8)
        s = lax.rem(wid, 8)
        wins = (win0_v, win1_v)

        def win_in_descs(w, slot):
            lanes = min(n_full - w * W, W)
            ds = [
                pltpu.make_async_copy(
                    table_hbm.at[pl.ds(g, 1), pl.ds(s, 1), pl.ds(w * W, lanes)],
                    wins[slot].at[:, :, pl.ds(0, lanes)],
                    sem,
                )
            ]
            if lanes < W:
                ds.append(
                    pltpu.make_async_copy(
                        tail_hbm.at[pl.ds(g, 1), pl.ds(s, 1), :],
                        wins[slot].at[:, :, pl.ds(lanes, _L)],
                        sem,
                    )
                )
            return ds

        pltpu.make_async_copy(idx_hbm, idx_v, idx_sem).start()
        first = win_in_descs(0, 0)
        for d in first:
            d.start()
        pltpu.make_async_copy(idx_hbm, idx_v, idx_sem).wait()

        zl = jnp.zeros((L,), jnp.int32)
        ones = jnp.ones((L,), jnp.int32)
        hist_v[pl.ds(0, L)] = zl
        hist_v[pl.ds(L, L)] = zl

        def hist_body(c4, _):
            bs = []
            for u in range(4):
                v = idx_v[pl.ds((c4 * 4 + u) * L, L)]
                bs.append(jax.lax.shift_right_logical(v, LOGW))
            for b in bs:
                plsc.addupdate_scatter(hist_v, [b], ones)
            return 0

        lax.fori_loop(0, n_chunks // 4, hist_body, 0)

        c0 = hist_v[pl.ds(0, L)]
        s0 = plsc.cumsum(c0)
        e0 = s0 - c0
        starts_v[pl.ds(0, L)] = e0
        cur_v[pl.ds(0, L)] = e0
        carry = jnp.sum(c0)
        c1 = hist_v[pl.ds(L, L)]
        s1 = plsc.cumsum(c1) + carry
        e1 = s1 - c1
        starts_v[pl.ds(L, L)] = e1
        cur_v[pl.ds(L, L)] = e1

        iota = lax.iota(jnp.int32, L)

        def _extract(vec, lane):
            return jnp.sum(jnp.where(iota == lane, vec, 0))

        def scat_body(c4, _):
            pre = []
            for u in range(4):
                c = c4 * 4 + u
                v = idx_v[pl.ds(c * L, L)]
                p = iota + c * L
                b = jax.lax.shift_right_logical(v, LOGW)
                cnt, last = plsc.scan_count(b)
                lidx = v - jax.lax.shift_left(b, LOGW)
                pk = jax.lax.shift_left(lidx, 14) + p
                pre.append((b, cnt, last, pk))
            for b, cnt, last, pk in pre:
                base = plsc.load_gather(cur_v, [b])
                plsc.store_scatter(pk_v, [base + cnt - 1], pk)
                plsc.addupdate_scatter(cur_v, [b], cnt, mask=last)
            return 0

        lax.fori_loop(0, n_chunks // 4, scat_body, 0)

        zero16 = jnp.zeros((L,), jnp.int32)
        for w in range(n_win):
            if w + 1 < n_win:
                for d in win_in_descs(w + 1, (w + 1) % 2):
                    d.start()
            for d in win_in_descs(w, w % 2):
                d.wait()
            lo = _extract(starts_v[pl.ds((w // L) * L, L)], w % L)
            hi = _extract(starts_v[pl.ds(((w + 1) // L) * L, L)], (w + 1) % L)
            k0 = jax.lax.shift_right_logical(lo, 4)
            k1 = jax.lax.shift_right_logical(hi + (L - 1), 4)

            def gat_body(c, _, w=w):
                base = c * L
                pk = pk_v[pl.ds(pl.multiple_of(base, L), L)]
                pos_i = iota + base
                m = jnp.logical_and(pos_i >= lo, pos_i < hi)
                lidx = jax.lax.shift_right_logical(pk, 14)
                p = jax.lax.bitwise_and(pk, (1 << 14) - 1)
                vals = plsc.load_gather(
                    wins[w % 2], [zero16, zero16, lidx], mask=m
                )
                plsc.store_scatter(row_v, [zero16, p], vals, mask=m)
                return 0

            lax.fori_loop(k0, k1, gat_body, 0)

        pltpu.sync_copy(row_v, out_hbm.at[pl.ds(wid, 1)])

    return k


@jax.jit
def kernel(id_indices, embeddings):
    B = id_indices.shape[0]
    V, D = embeddings.shape
    n_full = (V // _L) * _L

    table3 = embeddings.T.reshape(D // 8, 8, V)
    tail = embeddings[n_full:, :]
    tail_p = jnp.pad(tail.T, ((0, 0), (0, _L - tail.shape[0])))
    tail3 = tail_p.reshape(D // 8, 8, _L)

    k = _make_fused(V, D, B)
    out_t = k(table3, tail3, id_indices.astype(jnp.int32))
    return out_t.T
